# baseline (speedup 1.0000x reference)
import functools

import jax
import jax.numpy as jnp
from jax import lax
from jax.experimental import pallas as pl
from jax.experimental.pallas import tpu as pltpu
from jax.experimental.pallas import tpu_sc as plsc

_NC = 2
_NS = 16
_L = 16


def _sc_gather(flat_logits, x, B, V):
  mesh = plsc.VectorSubcoreMesh(core_axis_name="c", subcore_axis_name="s")

  @functools.partial(
      pl.kernel,
      out_type=jax.ShapeDtypeStruct((B,), jnp.float32),
      mesh=mesh,
      scratch_types=[
          pltpu.VMEM((B,), jnp.int32),
          pltpu.VMEM((B,), jnp.int32),
          pltpu.VMEM((B,), jnp.float32),
          pltpu.SemaphoreType.DMA,
      ],
  )
  def gather_k(flat_hbm, x_hbm, out_hbm, xv, idxv, pv, sem):
    wid = lax.axis_index("s") * _NC + lax.axis_index("c")

    @pl.when(wid == 0)
    def _():
      pltpu.sync_copy(x_hbm, xv)
      for i in range(B // _L):
        row = lax.iota(jnp.int32, _L) + (i * _L)
        idxv[pl.ds(i * _L, _L)] = row * V + xv[pl.ds(i * _L, _L)]
      pltpu.async_copy(flat_hbm.at[idxv], pv, sem).wait()
      pltpu.sync_copy(pv, out_hbm)

  return gather_k(flat_logits, x)


def _tc_body(nsplit, *refs):
  chunk_refs = refs[:nsplit]
  picked_ref, out_ref = refs[nsplit], refs[nsplit + 1]
  j = pl.program_id(0)

  chunks = [r[...][:, 0, 0, :] for r in chunk_refs]
  m = chunks[0].max(axis=1, keepdims=True)
  for c in chunks[1:]:
    m = jnp.maximum(m, c.max(axis=1, keepdims=True))
  s = jnp.exp(chunks[0] - m).sum(axis=1, keepdims=True)
  for c in chunks[1:]:
    s += jnp.exp(c - m).sum(axis=1, keepdims=True)
  part = -jnp.sum(m + jnp.log(s))

  @pl.when(j == 0)
  def _():
    out_ref[...] = (jnp.sum(picked_ref[...]) + part).reshape(1, 1)

  @pl.when(j > 0)
  def _():
    out_ref[...] += part.reshape(1, 1)


def kernel(logits, x):
  B, V = logits.shape
  x = x.astype(jnp.int32)

  picked = _sc_gather(logits.reshape(-1), x, B, V)

  R = 8
  S = 4
  C = V // S
  logits4 = logits.reshape(B, S, 1, C)
  in_specs = [
      pl.BlockSpec((R, 1, 1, C), functools.partial(lambda p, j: (j, p, 0, 0), p))
      for p in range(S)
  ]
  in_specs.append(pl.BlockSpec((1, B), lambda j: (0, 0)))
  out = pl.pallas_call(
      functools.partial(_tc_body, S),
      grid=(B // R,),
      in_specs=in_specs,
      out_specs=pl.BlockSpec((1, 1), lambda j: (0, 0)),
      out_shape=jax.ShapeDtypeStruct((1, 1), jnp.float32),
  )(*([logits4] * S), picked.reshape(1, B))
  return out[0, 0]


# speedup vs baseline: 1.6317x; 1.6317x over previous
import functools

import jax
import jax.numpy as jnp
from jax import lax
from jax.experimental import pallas as pl
from jax.experimental.pallas import tpu as pltpu
from jax.experimental.pallas import tpu_sc as plsc

_NC = 2
_NS = 16
_L = 16


def _sc_gather(flat_logits, x, B, V):
  mesh = plsc.VectorSubcoreMesh(core_axis_name="c", subcore_axis_name="s")

  @functools.partial(
      pl.kernel,
      out_type=jax.ShapeDtypeStruct((B,), jnp.float32),
      mesh=mesh,
      scratch_types=[
          pltpu.VMEM((B,), jnp.int32),
          pltpu.VMEM((B,), jnp.int32),
          pltpu.VMEM((B,), jnp.float32),
          pltpu.SemaphoreType.DMA,
      ],
  )
  def gather_k(flat_hbm, x_hbm, out_hbm, xv, idxv, pv, sem):
    wid = lax.axis_index("s") * _NC + lax.axis_index("c")

    @pl.when(wid == 0)
    def _():
      pltpu.sync_copy(x_hbm, xv)
      for i in range(B // _L):
        row = lax.iota(jnp.int32, _L) + (i * _L)
        idxv[pl.ds(i * _L, _L)] = row * V + xv[pl.ds(i * _L, _L)]
      pltpu.async_copy(flat_hbm.at[idxv], pv, sem).wait()
      pltpu.sync_copy(pv, out_hbm)

  return gather_k(flat_logits, x)


def _tc_body(B, R, NB, logits_hbm, picked_ref, out_ref, *scratch):
  nstripes = B // R
  bufs = scratch[:NB]
  sems = scratch[NB]

  def stripe_copy(i, b):
    return pltpu.make_async_copy(
        logits_hbm.at[pl.ds(i * R, R), :], bufs[b], sems.at[b])

  for b in range(min(NB, nstripes)):
    stripe_copy(b, b).start()

  total = jnp.zeros((1, 1), jnp.float32)
  for i in range(nstripes):
    b = i % NB
    stripe_copy(i, b).wait()
    chunk = bufs[b][...]
    m = chunk.max(axis=1, keepdims=True)
    s = jnp.exp(chunk - m).sum(axis=1, keepdims=True)
    if i + NB < nstripes:
      stripe_copy(i + NB, b).start()
    total = total - jnp.sum(m + jnp.log(s)).reshape(1, 1)

  out_ref[...] = total + jnp.sum(picked_ref[...]).reshape(1, 1)


def kernel(logits, x):
  B, V = logits.shape
  x = x.astype(jnp.int32)

  picked = _sc_gather(logits.reshape(-1), x, B, V)

  R = 8
  NB = 8
  out = pl.pallas_call(
      functools.partial(_tc_body, B, R, NB),
      in_specs=[
          pl.BlockSpec(memory_space=pltpu.MemorySpace.HBM),
          pl.BlockSpec((1, B), lambda: (0, 0)),
      ],
      out_specs=pl.BlockSpec((1, 1), lambda: (0, 0)),
      out_shape=jax.ShapeDtypeStruct((1, 1), jnp.float32),
      scratch_shapes=(
          [pltpu.VMEM((R, V), jnp.float32) for _ in range(NB)]
          + [pltpu.SemaphoreType.DMA((NB,))]
      ),
  )(logits, picked.reshape(1, B))
  return out[0, 0]


# speedup vs baseline: 1.6325x; 1.0005x over previous
import functools

import jax
import jax.numpy as jnp
from jax import lax
from jax.experimental import pallas as pl
from jax.experimental.pallas import tpu as pltpu
from jax.experimental.pallas import tpu_sc as plsc

_NC = 2
_NS = 16
_L = 16


def _sc_gather_sum(flat_logits, x, B, V):
  mesh = plsc.VectorSubcoreMesh(core_axis_name="c", subcore_axis_name="s")

  @functools.partial(
      pl.kernel,
      out_type=jax.ShapeDtypeStruct((_L,), jnp.float32),
      mesh=mesh,
      scratch_types=[
          pltpu.VMEM((B,), jnp.int32),
          pltpu.VMEM((B,), jnp.int32),
          pltpu.VMEM((B,), jnp.float32),
          pltpu.VMEM((_L,), jnp.float32),
          pltpu.SemaphoreType.DMA,
      ],
  )
  def gather_k(flat_hbm, x_hbm, out_hbm, xv, idxv, pv, sv, sem):
    wid = lax.axis_index("s") * _NC + lax.axis_index("c")

    @pl.when(wid == 0)
    def _():
      pltpu.sync_copy(x_hbm, xv)
      for i in range(B // _L):
        row = lax.iota(jnp.int32, _L) + (i * _L)
        idxv[pl.ds(i * _L, _L)] = row * V + xv[pl.ds(i * _L, _L)]
      pltpu.async_copy(flat_hbm.at[idxv], pv, sem).wait()
      acc = pv[pl.ds(0, _L)]
      for i in range(1, B // _L):
        acc = acc + pv[pl.ds(i * _L, _L)]
      sv[...] = acc
      pltpu.sync_copy(sv, out_hbm)

  return gather_k(flat_logits, x)


def _tc_body(B, R, NB, logits_hbm, out_ref, *scratch):
  nstripes = B // R
  bufs = scratch[:NB]
  sems = scratch[NB]

  def stripe_copy(i, b):
    return pltpu.make_async_copy(
        logits_hbm.at[pl.ds(i * R, R), :], bufs[b], sems.at[b])

  for b in range(min(NB, nstripes)):
    stripe_copy(b, b).start()

  total = jnp.zeros((1, 1), jnp.float32)
  for i in range(nstripes):
    b = i % NB
    stripe_copy(i, b).wait()
    chunk = bufs[b][...]
    m = chunk.max(axis=1, keepdims=True)
    s = jnp.exp(chunk - m).sum(axis=1, keepdims=True)
    if i + NB < nstripes:
      stripe_copy(i + NB, b).start()
    total = total - jnp.sum(m + jnp.log(s)).reshape(1, 1)

  out_ref[...] = total


def kernel(logits, x):
  B, V = logits.shape
  x = x.astype(jnp.int32)

  picked_sum = _sc_gather_sum(logits.reshape(-1), x, B, V)

  R = 8
  NB = 8
  neg_lse_sum = pl.pallas_call(
      functools.partial(_tc_body, B, R, NB),
      in_specs=[pl.BlockSpec(memory_space=pltpu.MemorySpace.HBM)],
      out_specs=pl.BlockSpec((1, 1), lambda: (0, 0)),
      out_shape=jax.ShapeDtypeStruct((1, 1), jnp.float32),
      scratch_shapes=(
          [pltpu.VMEM((R, V), jnp.float32) for _ in range(NB)]
          + [pltpu.SemaphoreType.DMA((NB,))]
      ),
  )(logits)
  return neg_lse_sum[0, 0] + jnp.sum(picked_sum)


# speedup vs baseline: 3.2821x; 2.0104x over previous
import functools

import jax
import jax.numpy as jnp
from jax import lax
from jax.experimental import pallas as pl
from jax.experimental.pallas import tpu as pltpu


def _tc_body(B, V, R, NB, logits_hbm, x_ref, out_ref, *scratch):
  nstripes = B // R
  bufs = scratch[:NB]
  sems = scratch[NB]

  def stripe_copy(i, b):
    return pltpu.make_async_copy(
        logits_hbm.at[pl.ds(i * R, R), :], bufs[b], sems.at[b])

  for b in range(min(NB, nstripes)):
    stripe_copy(b, b).start()

  total = jnp.zeros((1, 1), jnp.float32)
  for i in range(nstripes):
    b = i % NB
    stripe_copy(i, b).wait()
    chunk = bufs[b][...]
    xrows = x_ref[0, i * R:(i + 1) * R].reshape(R, 1)
    col = lax.broadcasted_iota(jnp.int32, (R, V), 1)
    picked = jnp.where(col == xrows, chunk, 0.0).sum(axis=1, keepdims=True)
    m = chunk.max(axis=1, keepdims=True)
    s = jnp.exp(chunk - m).sum(axis=1, keepdims=True)
    if i + NB < nstripes:
      stripe_copy(i + NB, b).start()
    total = total + jnp.sum(picked - m - jnp.log(s)).reshape(1, 1)

  out_ref[...] = total


def kernel(logits, x):
  B, V = logits.shape
  x = x.astype(jnp.int32)

  R = 8
  NB = 8
  out = pl.pallas_call(
      functools.partial(_tc_body, B, V, R, NB),
      in_specs=[
          pl.BlockSpec(memory_space=pltpu.MemorySpace.HBM),
          pl.BlockSpec((1, B), lambda: (0, 0)),
      ],
      out_specs=pl.BlockSpec((1, 1), lambda: (0, 0)),
      out_shape=jax.ShapeDtypeStruct((1, 1), jnp.float32),
      scratch_shapes=(
          [pltpu.VMEM((R, V), jnp.float32) for _ in range(NB)]
          + [pltpu.SemaphoreType.DMA((NB,))]
      ),
  )(logits, x.reshape(1, B))
  return out[0, 0]


# speedup vs baseline: 3.2918x; 1.0030x over previous
import functools

import jax
import jax.numpy as jnp
from jax import lax
from jax.experimental import pallas as pl
from jax.experimental.pallas import tpu as pltpu


def _tc_body(B, V, R, NB, logits_hbm, x_ref, out_ref, *scratch):
  nstripes = B // R
  bufs = scratch[:NB]
  sems = scratch[NB]

  def stripe_copy(i, b):
    return pltpu.make_async_copy(
        logits_hbm.at[pl.ds(i * R, R), :], bufs[b], sems.at[b])

  for b in range(min(NB, nstripes)):
    stripe_copy(b, b).start()

  total = jnp.zeros((1, 1), jnp.float32)
  for i in range(nstripes):
    b = i % NB
    stripe_copy(i, b).wait()
    chunk = bufs[b][...]
    xrows = x_ref[0, i * R:(i + 1) * R].reshape(R, 1)
    col = lax.broadcasted_iota(jnp.int32, (R, V), 1)
    picked = jnp.where(col == xrows, chunk, 0.0).sum(axis=1, keepdims=True)
    m = chunk.max(axis=1, keepdims=True)
    s = jnp.exp(chunk - m).sum(axis=1, keepdims=True)
    if i + NB < nstripes:
      stripe_copy(i + NB, b).start()
    total = total + jnp.sum(picked - m - jnp.log(s)).reshape(1, 1)

  out_ref[...] = total


def kernel(logits, x):
  B, V = logits.shape
  x = x.astype(jnp.int32)

  R = 8
  NB = 16
  out = pl.pallas_call(
      functools.partial(_tc_body, B, V, R, NB),
      in_specs=[
          pl.BlockSpec(memory_space=pltpu.MemorySpace.HBM),
          pl.BlockSpec((1, B), lambda: (0, 0)),
      ],
      out_specs=pl.BlockSpec((1, 1), lambda: (0, 0)),
      out_shape=jax.ShapeDtypeStruct((1, 1), jnp.float32),
      scratch_shapes=(
          [pltpu.VMEM((R, V), jnp.float32) for _ in range(NB)]
          + [pltpu.SemaphoreType.DMA((NB,))]
      ),
  )(logits, x.reshape(1, B))
  return out[0, 0]


# speedup vs baseline: 3.8949x; 1.1832x over previous
import functools

import jax
import jax.numpy as jnp
from jax import lax
from jax.experimental import pallas as pl
from jax.experimental.pallas import tpu as pltpu


def _tc_body(B, V, R, NB, logits_hbm, x_ref, out_ref, *scratch):
  nstripes = B // R
  bufs = scratch[:NB]
  sems = scratch[NB]

  def stripe_copy(i, b):
    return pltpu.make_async_copy(
        logits_hbm.at[pl.ds(i * R, R), :], bufs[b], sems.at[b])

  for b in range(min(NB, nstripes)):
    stripe_copy(b, b).start()

  total = jnp.zeros((1, 1), jnp.float32)
  for i in range(nstripes):
    b = i % NB
    stripe_copy(i, b).wait()
    chunk = bufs[b][...]
    xrows = x_ref[0, i * R:(i + 1) * R].reshape(R, 1)
    col = lax.broadcasted_iota(jnp.int32, (R, V), 1)
    picked = jnp.where(col == xrows, chunk, 0.0).sum(axis=1, keepdims=True)
    m = chunk.max(axis=1, keepdims=True)
    s = jnp.exp(chunk - m).sum(axis=1, keepdims=True)
    if i + NB < nstripes:
      stripe_copy(i + NB, b).start()
    total = total + jnp.sum(picked - m - jnp.log(s)).reshape(1, 1)

  out_ref[...] = total


def kernel(logits, x):
  B, V = logits.shape
  x = x.astype(jnp.int32)

  R = 32
  NB = 4
  out = pl.pallas_call(
      functools.partial(_tc_body, B, V, R, NB),
      in_specs=[
          pl.BlockSpec(memory_space=pltpu.MemorySpace.HBM),
          pl.BlockSpec((1, B), lambda: (0, 0)),
      ],
      out_specs=pl.BlockSpec((1, 1), lambda: (0, 0)),
      out_shape=jax.ShapeDtypeStruct((1, 1), jnp.float32),
      scratch_shapes=(
          [pltpu.VMEM((R, V), jnp.float32) for _ in range(NB)]
          + [pltpu.SemaphoreType.DMA((NB,))]
      ),
  )(logits, x.reshape(1, B))
  return out[0, 0]
